# baseline (speedup 1.0000x reference)
import functools

import jax
import jax.numpy as jnp
from jax import lax
from jax.experimental import pallas as pl
from jax.experimental.pallas import tpu as pltpu
from jax.experimental.pallas import tpu_sc as plsc

N = 10000
D = 128
C = 2
NPAD = 10240
CHUNK = 128
NCORES = 2
NSUB = 16
NTILES = NCORES * NSUB
ROWS_PER_TILE = NPAD // NSUB
TRASH = NPAD - 1


def _make_seg_sum(nchunk):
    mesh = plsc.VectorSubcoreMesh(core_axis_name="c", subcore_axis_name="s",
                                  num_cores=NCORES, num_subcores=NSUB)

    nib = 8
    nblk = nchunk // nib

    @functools.partial(
        pl.kernel,
        mesh=mesh,
        out_type=jax.ShapeDtypeStruct((2 * NPAD, D), jnp.float32),
        scratch_types=[
            pltpu.VMEM((nib, CHUNK), jnp.int32),
            pltpu.VMEM((nib, CHUNK), jnp.int32),
            pltpu.VMEM((CHUNK, D), jnp.float32),
            pltpu.VMEM((CHUNK, D), jnp.float32),
            pltpu.VMEM_SHARED((NPAD, D), jnp.float32),
            pltpu.SemaphoreType.DMA,
            pltpu.SemaphoreType.DMA,
        ],
    )
    def seg_sum(x_hbm, src_hbm, dst_hbm, zeros_hbm, out_hbm,
                src_v, dst_v, rows0_v, rows1_v, agg_sh, sem0, sem1):
        c = lax.axis_index("c")
        s = lax.axis_index("s")
        wid = c * NSUB + s

        pltpu.sync_copy(zeros_hbm, agg_sh.at[pl.ds(s * ROWS_PER_TILE,
                                                   ROWS_PER_TILE)])
        plsc.subcore_barrier()

        row0 = wid * nchunk

        half = CHUNK // 2

        def gather(j, buf, sem):
            pltpu.async_copy(x_hbm.at[src_v.at[j, pl.ds(0, half)]],
                             buf.at[pl.ds(0, half)], sem)
            pltpu.async_copy(x_hbm.at[src_v.at[j, pl.ds(half, half)]],
                             buf.at[pl.ds(half, half)], sem)

        def gwait(j, buf, sem):
            pltpu.make_async_copy(x_hbm.at[src_v.at[j, pl.ds(0, half)]],
                                  buf.at[pl.ds(0, half)], sem).wait()
            pltpu.make_async_copy(x_hbm.at[src_v.at[j, pl.ds(half, half)]],
                                  buf.at[pl.ds(half, half)], sem).wait()

        def block(bi, _):
            pltpu.sync_copy(src_hbm.at[pl.ds(row0 + bi * nib, nib)], src_v)
            pltpu.sync_copy(dst_hbm.at[pl.ds(row0 + bi * nib, nib)], dst_v)
            bufs = (rows0_v, rows1_v)
            sems = (sem0, sem1)
            gather(0, rows0_v, sem0)
            for j in range(nib):
                buf, sem = bufs[j % 2], sems[j % 2]
                gwait(j, buf, sem)
                if j + 1 < nib:
                    gather(j + 1, bufs[(j + 1) % 2], sems[(j + 1) % 2])
                pltpu.sync_copy(buf, agg_sh.at[dst_v.at[j]], add=True)
            return 0

        lax.fori_loop(0, nblk, block, 0)
        plsc.subcore_barrier()

        out_base = c * NPAD + s * ROWS_PER_TILE
        for k in range(ROWS_PER_TILE // CHUNK):
            pltpu.sync_copy(agg_sh.at[pl.ds(s * ROWS_PER_TILE + k * CHUNK,
                                            CHUNK)], rows0_v)
            pltpu.sync_copy(rows0_v, out_hbm.at[pl.ds(out_base + k * CHUNK,
                                                      CHUNK)])

    return seg_sum


def _mlp_body(x_ref, a0_ref, a1_ref, w1_ref, b1_ref, w2_ref, b2_ref, o_ref):
    h = x_ref[...] + a0_ref[...] + a1_ref[...]
    t = jnp.maximum(
        jnp.dot(h, w1_ref[...], preferred_element_type=jnp.float32)
        + b1_ref[...], 0.0)
    o = jnp.dot(t, w2_ref[...], preferred_element_type=jnp.float32) \
        + b2_ref[...]
    o_ref[...] = jnp.maximum(o, 0.0)


def _mlp(x, a0, a1, w1, b1, w2, b2):
    bm = 1024
    grid = NPAD // bm
    row = lambda i: (i, 0)
    full = lambda i: (0, 0)
    return pl.pallas_call(
        _mlp_body,
        grid=(grid,),
        in_specs=[
            pl.BlockSpec((bm, D), row),
            pl.BlockSpec((bm, D), row),
            pl.BlockSpec((bm, D), row),
            pl.BlockSpec((D, D), full),
            pl.BlockSpec((1, D), full),
            pl.BlockSpec((D, D), full),
            pl.BlockSpec((1, D), full),
        ],
        out_specs=pl.BlockSpec((bm, D), row),
        out_shape=jax.ShapeDtypeStruct((NPAD, D), jnp.float32),
    )(x, a0, a1, w1, b1, w2, b2)


def _att_body(x_ref, a0_ref, a1_ref, w1_ref, b1_ref, w2_ref, b2_ref,
              wa_ref, ba_ref, wb_ref, bb_ref, wcb_ref, bc_ref,
              wr_ref, br_ref, wcls_ref, bcls_ref,
              s_out_ref, logit_ref, prob_ref):
    f32 = jnp.float32
    h = x_ref[...] + a0_ref[...] + a1_ref[...]
    t = jnp.maximum(
        jnp.dot(h, w1_ref[...], preferred_element_type=f32) + b1_ref[...],
        0.0)
    h3 = jnp.maximum(
        jnp.dot(t, w2_ref[...], preferred_element_type=f32) + b2_ref[...],
        0.0)
    a = jnp.tanh(jnp.dot(h3, wa_ref[...], preferred_element_type=f32)
                 + ba_ref[...])
    b = jax.nn.sigmoid(jnp.dot(h3, wb_ref[...], preferred_element_type=f32)
                       + bb_ref[...])
    s_mat = jnp.dot(a * b, wcb_ref[...], preferred_element_type=f32) \
        + bc_ref[...]
    s_out_ref[...] = s_mat
    rows = lax.broadcasted_iota(jnp.int32, (NPAD, D), 0)
    valid = rows < N
    m = jnp.max(jnp.where(valid, s_mat, -1e30))
    p = jnp.where(valid, jnp.exp(s_mat - m), 0.0)
    z = jnp.sum(p) * (1.0 / D)
    pooled = jnp.sum(h3 * p, axis=0, keepdims=True) / z
    h2 = jnp.maximum(
        jnp.dot(pooled, wr_ref[...], preferred_element_type=f32)
        + br_ref[...], 0.0)
    logits = jnp.dot(h2, wcls_ref[...], preferred_element_type=f32) \
        + bcls_ref[...]
    logit_ref[...] = jnp.broadcast_to(logits, (8, D))
    prob_ref[...] = jnp.broadcast_to(jax.nn.sigmoid(logits), (8, D))


def _att(x, a0, a1, w1, b1, w2, b2, wa, ba, wb, bb, wcb, bc, wr, br,
         wcls, bcls):
    full = lambda: (0, 0)
    spec = lambda shape: pl.BlockSpec(shape, lambda: (0, 0))
    return pl.pallas_call(
        _att_body,
        in_specs=[
            spec((NPAD, D)), spec((NPAD, D)), spec((NPAD, D)),
            spec((D, D)), spec((1, D)), spec((D, D)), spec((1, D)),
            spec((D, D)), spec((1, D)), spec((D, D)), spec((1, D)),
            spec((D, D)), spec((1, D)),
            spec((D, D)), spec((1, D)), spec((D, D)), spec((1, D)),
        ],
        out_specs=[spec((NPAD, D)), spec((8, D)), spec((8, D))],
        out_shape=[
            jax.ShapeDtypeStruct((NPAD, D), jnp.float32),
            jax.ShapeDtypeStruct((8, D), jnp.float32),
            jax.ShapeDtypeStruct((8, D), jnp.float32),
        ],
    )(x, a0, a1, w1, b1, w2, b2, wa, ba, wb, bb, wcb, bc, wr, br, wcls, bcls)


def kernel(x, edge_index, label, W1a, b1a, W2a, b2a, W1b, b1b, W2b, b2b,
           W1c, b1c, W2c, b2c, Wa, ba, Wb, bb, Wc, bc, Wr, br, Wcls, bcls):
    e = edge_index.shape[1]
    nchunk = -(-e // (NTILES * CHUNK * 8)) * 8
    ept = nchunk * CHUNK
    epad = ept * NTILES
    pad = epad - e

    src = jnp.concatenate(
        [edge_index[0], jnp.zeros((pad,), jnp.int32)]).reshape(-1, CHUNK)
    trash = N + jnp.arange(pad, dtype=jnp.int32) % (NPAD - N)
    dst = jnp.concatenate([edge_index[1], trash]).reshape(-1, CHUNK)
    zeros = jnp.zeros((ROWS_PER_TILE, D), jnp.float32)
    xp = jnp.zeros((NPAD, D), jnp.float32).at[:N, :].set(x)

    seg = _make_seg_sum(nchunk)

    agg = seg(xp, src, dst, zeros)
    x1 = _mlp(xp, agg[:NPAD], agg[NPAD:], W1a, b1a[None], W2a, b2a[None])
    agg = seg(x1, src, dst, zeros)
    x2 = _mlp(x1, agg[:NPAD], agg[NPAD:], W1b, b1b[None], W2b, b2b[None])
    agg = seg(x2, src, dst, zeros)

    wcb = jnp.tile(Wc, (1, D))
    bcb = jnp.full((1, D), bc[0], jnp.float32)
    wclsp = jnp.zeros((D, D), jnp.float32).at[:, :C].set(Wcls)
    bclsp = jnp.zeros((1, D), jnp.float32).at[0, :C].set(bcls)

    scores, logit8, prob8 = _att(
        x2, agg[:NPAD], agg[NPAD:], W1c, b1c[None], W2c, b2c[None],
        Wa, ba[None], Wb, bb[None], wcb, bcb, Wr, br[None], wclsp, bclsp)

    A_path = scores[:N, 0][None, :]
    logits = logit8[0:1, :C]
    Y_prob = prob8[0:1, :C]
    return (logits, Y_prob, A_path, label)


# speedup vs baseline: 1.1091x; 1.1091x over previous
import functools

import jax
import jax.numpy as jnp
from jax import lax
from jax.experimental import pallas as pl
from jax.experimental.pallas import tpu as pltpu
from jax.experimental.pallas import tpu_sc as plsc

N = 10000
D = 128
C = 2
NPAD = 10240
CHUNK = 128
NCORES = 2
NSUB = 16
NTILES = NCORES * NSUB
ROWS_PER_TILE = NPAD // NSUB
TRASH = NPAD - 1


def _make_seg_sum(nchunk0, nchunk1):
    mesh = plsc.VectorSubcoreMesh(core_axis_name="c", subcore_axis_name="s",
                                  num_cores=NCORES, num_subcores=NSUB)

    nib = 8

    @functools.partial(
        pl.kernel,
        mesh=mesh,
        out_type=jax.ShapeDtypeStruct((2 * NPAD, D), jnp.float32),
        scratch_types=[
            pltpu.VMEM((nib, CHUNK), jnp.int32),
            pltpu.VMEM((nib, CHUNK), jnp.int32),
            pltpu.VMEM((CHUNK, D), jnp.float32),
            pltpu.VMEM((CHUNK, D), jnp.float32),
            pltpu.VMEM_SHARED((NPAD, D), jnp.float32),
            pltpu.SemaphoreType.DMA,
            pltpu.SemaphoreType.DMA,
        ],
    )
    def seg_sum(x_hbm, src_hbm, dst_hbm, zeros_hbm, out_hbm,
                src_v, dst_v, rows0_v, rows1_v, agg_sh, sem0, sem1):
        c = lax.axis_index("c")
        s = lax.axis_index("s")

        pltpu.sync_copy(zeros_hbm, agg_sh.at[pl.ds(s * ROWS_PER_TILE,
                                                   ROWS_PER_TILE)])
        plsc.subcore_barrier()

        row0 = jnp.where(c == 0, s * nchunk0,
                         NSUB * nchunk0 + s * nchunk1)
        nblk = jnp.where(c == 0, nchunk0 // nib, nchunk1 // nib)

        def block(bi, _):
            pltpu.sync_copy(src_hbm.at[pl.ds(row0 + bi * nib, nib)], src_v)
            pltpu.sync_copy(dst_hbm.at[pl.ds(row0 + bi * nib, nib)], dst_v)
            bufs = (rows0_v, rows1_v)
            sems = (sem0, sem1)
            pltpu.async_copy(x_hbm.at[src_v.at[0]], rows0_v, sem0)
            for j in range(nib):
                buf, sem = bufs[j % 2], sems[j % 2]
                pltpu.make_async_copy(x_hbm.at[src_v.at[j]], buf, sem).wait()
                if j + 1 < nib:
                    pltpu.async_copy(x_hbm.at[src_v.at[j + 1]],
                                     bufs[(j + 1) % 2], sems[(j + 1) % 2])
                pltpu.sync_copy(buf, agg_sh.at[dst_v.at[j]], add=True)
            return 0

        lax.fori_loop(0, nblk, block, 0)
        plsc.subcore_barrier()

        out_base = c * NPAD + s * ROWS_PER_TILE
        for k in range(ROWS_PER_TILE // CHUNK):
            pltpu.sync_copy(agg_sh.at[pl.ds(s * ROWS_PER_TILE + k * CHUNK,
                                            CHUNK)], rows0_v)
            pltpu.sync_copy(rows0_v, out_hbm.at[pl.ds(out_base + k * CHUNK,
                                                      CHUNK)])

    return seg_sum


def _mlp_body(x_ref, a0_ref, a1_ref, w1_ref, b1_ref, w2_ref, b2_ref, o_ref):
    h = x_ref[...] + a0_ref[...] + a1_ref[...]
    t = jnp.maximum(
        jnp.dot(h, w1_ref[...], preferred_element_type=jnp.float32)
        + b1_ref[...], 0.0)
    o = jnp.dot(t, w2_ref[...], preferred_element_type=jnp.float32) \
        + b2_ref[...]
    o_ref[...] = jnp.maximum(o, 0.0)


def _mlp(x, a0, a1, w1, b1, w2, b2):
    bm = 1024
    grid = NPAD // bm
    row = lambda i: (i, 0)
    full = lambda i: (0, 0)
    return pl.pallas_call(
        _mlp_body,
        grid=(grid,),
        in_specs=[
            pl.BlockSpec((bm, D), row),
            pl.BlockSpec((bm, D), row),
            pl.BlockSpec((bm, D), row),
            pl.BlockSpec((D, D), full),
            pl.BlockSpec((1, D), full),
            pl.BlockSpec((D, D), full),
            pl.BlockSpec((1, D), full),
        ],
        out_specs=pl.BlockSpec((bm, D), row),
        out_shape=jax.ShapeDtypeStruct((NPAD, D), jnp.float32),
    )(x, a0, a1, w1, b1, w2, b2)


def _att_body(x_ref, a0_ref, a1_ref, w1_ref, b1_ref, w2_ref, b2_ref,
              wa_ref, ba_ref, wb_ref, bb_ref, wcb_ref, bc_ref,
              wr_ref, br_ref, wcls_ref, bcls_ref,
              s_out_ref, logit_ref, prob_ref):
    f32 = jnp.float32
    h = x_ref[...] + a0_ref[...] + a1_ref[...]
    t = jnp.maximum(
        jnp.dot(h, w1_ref[...], preferred_element_type=f32) + b1_ref[...],
        0.0)
    h3 = jnp.maximum(
        jnp.dot(t, w2_ref[...], preferred_element_type=f32) + b2_ref[...],
        0.0)
    a = jnp.tanh(jnp.dot(h3, wa_ref[...], preferred_element_type=f32)
                 + ba_ref[...])
    b = jax.nn.sigmoid(jnp.dot(h3, wb_ref[...], preferred_element_type=f32)
                       + bb_ref[...])
    s_mat = jnp.dot(a * b, wcb_ref[...], preferred_element_type=f32) \
        + bc_ref[...]
    s_out_ref[...] = s_mat
    rows = lax.broadcasted_iota(jnp.int32, (NPAD, D), 0)
    valid = rows < N
    m = jnp.max(jnp.where(valid, s_mat, -1e30))
    p = jnp.where(valid, jnp.exp(s_mat - m), 0.0)
    z = jnp.sum(p) * (1.0 / D)
    pooled = jnp.sum(h3 * p, axis=0, keepdims=True) / z
    h2 = jnp.maximum(
        jnp.dot(pooled, wr_ref[...], preferred_element_type=f32)
        + br_ref[...], 0.0)
    logits = jnp.dot(h2, wcls_ref[...], preferred_element_type=f32) \
        + bcls_ref[...]
    logit_ref[...] = jnp.broadcast_to(logits, (8, D))
    prob_ref[...] = jnp.broadcast_to(jax.nn.sigmoid(logits), (8, D))


def _att(x, a0, a1, w1, b1, w2, b2, wa, ba, wb, bb, wcb, bc, wr, br,
         wcls, bcls):
    full = lambda: (0, 0)
    spec = lambda shape: pl.BlockSpec(shape, lambda: (0, 0))
    return pl.pallas_call(
        _att_body,
        in_specs=[
            spec((NPAD, D)), spec((NPAD, D)), spec((NPAD, D)),
            spec((D, D)), spec((1, D)), spec((D, D)), spec((1, D)),
            spec((D, D)), spec((1, D)), spec((D, D)), spec((1, D)),
            spec((D, D)), spec((1, D)),
            spec((D, D)), spec((1, D)), spec((D, D)), spec((1, D)),
        ],
        out_specs=[spec((NPAD, D)), spec((8, D)), spec((8, D))],
        out_shape=[
            jax.ShapeDtypeStruct((NPAD, D), jnp.float32),
            jax.ShapeDtypeStruct((8, D), jnp.float32),
            jax.ShapeDtypeStruct((8, D), jnp.float32),
        ],
    )(x, a0, a1, w1, b1, w2, b2, wa, ba, wb, bb, wcb, bc, wr, br, wcls, bcls)


def kernel(x, edge_index, label, W1a, b1a, W2a, b2a, W1b, b1b, W2b, b2b,
           W1c, b1c, W2c, b2c, Wa, ba, Wb, bb, Wc, bc, Wr, br, Wcls, bcls):
    e = edge_index.shape[1]
    nch_total = -(-e // (NTILES * CHUNK * 8)) * 8 * NTILES
    epad = nch_total * CHUNK
    pad = epad - e
    nchunk0 = (nch_total * 3 // 4) // (NSUB * 8) * 8
    nchunk1 = (nch_total - NSUB * nchunk0) // NSUB

    src = jnp.concatenate(
        [edge_index[0], jnp.zeros((pad,), jnp.int32)]).reshape(-1, CHUNK)
    trash = N + jnp.arange(pad, dtype=jnp.int32) % (NPAD - N)
    dst = jnp.concatenate([edge_index[1], trash]).reshape(-1, CHUNK)
    zeros = jnp.zeros((ROWS_PER_TILE, D), jnp.float32)
    xp = jnp.zeros((NPAD, D), jnp.float32).at[:N, :].set(x)

    assert NSUB * (nchunk0 + nchunk1) == nch_total and nchunk1 % 8 == 0
    seg = _make_seg_sum(nchunk0, nchunk1)

    agg = seg(xp, src, dst, zeros)
    x1 = _mlp(xp, agg[:NPAD], agg[NPAD:], W1a, b1a[None], W2a, b2a[None])
    agg = seg(x1, src, dst, zeros)
    x2 = _mlp(x1, agg[:NPAD], agg[NPAD:], W1b, b1b[None], W2b, b2b[None])
    agg = seg(x2, src, dst, zeros)

    wcb = jnp.tile(Wc, (1, D))
    bcb = jnp.full((1, D), bc[0], jnp.float32)
    wclsp = jnp.zeros((D, D), jnp.float32).at[:, :C].set(Wcls)
    bclsp = jnp.zeros((1, D), jnp.float32).at[0, :C].set(bcls)

    scores, logit8, prob8 = _att(
        x2, agg[:NPAD], agg[NPAD:], W1c, b1c[None], W2c, b2c[None],
        Wa, ba[None], Wb, bb[None], wcb, bcb, Wr, br[None], wclsp, bclsp)

    A_path = scores[:N, 0][None, :]
    logits = logit8[0:1, :C]
    Y_prob = prob8[0:1, :C]
    return (logits, Y_prob, A_path, label)
